# baseline (speedup 1.0000x reference)
import functools

import jax
import jax.numpy as jnp
from jax import lax
from jax.experimental import pallas as pl
from jax.experimental.pallas import tpu as pltpu
from jax.experimental.pallas import tpu_sc as plsc

VOCAB = 1000000
D = 32
B = 16384
NF = 26
FT = 2 * NF
L = 16

NC = 2
NS = 16
NW = NC * NS
BPW = B // NW

CHUNK = 2
ROWS = CHUNK * FT
NCH = BPW // CHUNK
GRPS = CHUNK * 2
INV = 1.0 / NF

_mesh = plsc.VectorSubcoreMesh(core_axis_name="c", subcore_axis_name="s")


@functools.partial(
    pl.kernel,
    mesh=_mesh,
    compiler_params=pltpu.CompilerParams(
        use_tc_tiling_on_sc=False, needs_layout_passes=False),
    out_type=jax.ShapeDtypeStruct((B * 2, D), jnp.float32),
    scratch_types=[
        pltpu.VMEM((NCH, ROWS), jnp.int32),
        pltpu.VMEM((ROWS, L), jnp.int32),
        pltpu.VMEM((ROWS, L), jnp.int32),
        pltpu.VMEM((BPW * 2, D), jnp.float32),
        pltpu.SemaphoreType.DMA,
        pltpu.SemaphoreType.DMA,
    ],
)
def _emb_pool(idx_hbm, table_hbm, out_hbm, idx_v, rows0, rows1, out_v,
              sem0, sem1):
    wid = lax.axis_index("s") * NC + lax.axis_index("c")
    base = wid * NCH

    pltpu.sync_copy(idx_hbm.at[pl.ds(base, NCH)], idx_v)

    bufs = (rows0, rows1)
    sems = (sem0, sem1)

    def gather(g, b, sem):
        return pltpu.make_async_copy(table_hbm.at[idx_v.at[g]], bufs[b], sem)

    gather(0, 0, sem0).start()
    gather(1, 1, sem1).start()

    hi_mask = jnp.full((L,), -65536, dtype=jnp.int32)

    def expand(words):
        ev = plsc.bitcast(lax.shift_left(words, 16), jnp.float32)
        od = plsc.bitcast(lax.bitwise_and(words, hi_mask), jnp.float32)
        return ev, od

    def reduce_chunk(g, buf):
        for grp in range(GRPS):
            s = grp * NF
            a_ev, a_od = expand(buf[s, 0:L])
            for j in range(1, NF):
                ev, od = expand(buf[s + j, 0:L])
                a_ev = a_ev + ev
                a_od = a_od + od
            orow = g * GRPS + grp
            out_v[orow, 0:L] = a_ev * INV
            out_v[orow, L:D] = a_od * INV

    def body(i, carry):
        for b in range(2):
            g = 2 * i + b
            gather(g, b, sems[b]).wait()
            reduce_chunk(g, bufs[b])

            @pl.when(g < NCH - 2)
            def _():
                gather(g + 2, b, sems[b]).start()

        return carry

    lax.fori_loop(0, NCH // 2, body, 0)

    pltpu.sync_copy(out_v, out_hbm.at[pl.ds(base * GRPS, BPW * 2)])


def kernel(x1, x2, table):
    idx = jnp.concatenate(
        [x1.astype(jnp.int32), x2.astype(jnp.int32)], axis=1)
    idx = idx.reshape(B // CHUNK, ROWS)
    tblw = jax.lax.bitcast_convert_type(
        table.astype(jnp.bfloat16).reshape(VOCAB, L, 2), jnp.int32)
    out = _emb_pool(idx, tblw)
    out = jnp.stack([out[:, :L], out[:, L:]], axis=-1)
    return out.reshape(B, 2, D)


# speedup vs baseline: 1.9035x; 1.9035x over previous
import functools

import jax
import jax.numpy as jnp
from jax import lax
from jax.experimental import pallas as pl
from jax.experimental.pallas import tpu as pltpu
from jax.experimental.pallas import tpu_sc as plsc

VOCAB = 1000000
D = 32
B = 16384
NF = 26
FT = 2 * NF
L = 16

NC = 2
NS = 16
NW = NC * NS
BPW = B // NW

BLK = 2048
NBLK = VOCAB // BLK
NVB = BLK // L
TAIL = 512
NTT = VOCAB - NBLK * BLK - TAIL

CHUNK = 2
ROWS = CHUNK * FT
NCH = BPW // CHUNK
GRPS = CHUNK * 2
INV = 1.0 / NF

_mesh = plsc.VectorSubcoreMesh(core_axis_name="c", subcore_axis_name="s")


@functools.partial(
    pl.kernel,
    mesh=_mesh,
    compiler_params=pltpu.CompilerParams(
        use_tc_tiling_on_sc=True, needs_layout_passes=False),
    out_type=jax.ShapeDtypeStruct((VOCAB * D,), jnp.float32),
    scratch_types=[
        pltpu.VMEM((D * BLK,), jnp.float32),
        pltpu.VMEM((BLK * D,), jnp.float32),
        pltpu.SemaphoreType.DMA,
        pltpu.SemaphoreType.DMA,
    ],
)
def _detranspose(tt_hbm, tail_hbm, dense_hbm, stage, outb, sin, sout):
    wid = lax.axis_index("s") * NC + lax.axis_index("c")
    base = wid * 15 + jnp.minimum(wid, 8)
    nblk = jnp.where(wid < 8, 16, 15)

    iota32 = lax.iota(jnp.int32, L) * D

    def do_block(i0, nv):
        w = nv * L
        for d in range(D):
            pltpu.make_async_copy(
                tt_hbm.at[d, pl.ds(i0, w)],
                stage.at[pl.ds(d * BLK, w)], sin).start()
        for d in range(D):
            pltpu.make_async_copy(
                tt_hbm.at[d, pl.ds(i0, w)],
                stage.at[pl.ds(d * BLK, w)], sin).wait()

        def vgroup(v, carry):
            av = iota32 + v * (L * D)
            for d in range(D):
                x = stage[pl.ds(d * BLK + v * L, L)]
                plsc.store_scatter(outb, [av + d], x)
            return carry

        lax.fori_loop(0, nv, vgroup, 0)
        pltpu.make_async_copy(
            outb.at[pl.ds(0, nv * L * D)],
            dense_hbm.at[pl.ds(i0 * D, nv * L * D)], sout).start()
        pltpu.make_async_copy(
            outb.at[pl.ds(0, nv * L * D)],
            dense_hbm.at[pl.ds(i0 * D, nv * L * D)], sout).wait()

    def body(k, carry):
        do_block((base + k) * BLK, NVB)
        return carry

    lax.fori_loop(0, nblk, body, 0)

    @pl.when(wid == 31)
    def _():
        do_block(NBLK * BLK, TAIL // L)

    @pl.when(wid == 30)
    def _():
        pltpu.sync_copy(
            tail_hbm, dense_hbm.at[pl.ds((NBLK * BLK + TAIL) * D, NTT * D)])


@functools.partial(
    pl.kernel,
    mesh=_mesh,
    compiler_params=pltpu.CompilerParams(use_tc_tiling_on_sc=False),
    out_type=jax.ShapeDtypeStruct((B * 2, D), jnp.float32),
    scratch_types=[
        pltpu.VMEM((NCH, ROWS), jnp.int32),
        pltpu.VMEM((ROWS, D), jnp.float32),
        pltpu.VMEM((ROWS, D), jnp.float32),
        pltpu.VMEM((BPW * 2, D), jnp.float32),
        pltpu.SemaphoreType.DMA,
        pltpu.SemaphoreType.DMA,
    ],
)
def _emb_pool(idx_hbm, table_hbm, out_hbm, idx_v, rows0, rows1, out_v,
              sem0, sem1):
    wid = lax.axis_index("s") * NC + lax.axis_index("c")
    base = wid * NCH

    pltpu.sync_copy(idx_hbm.at[pl.ds(base, NCH)], idx_v)

    bufs = (rows0, rows1)
    sems = (sem0, sem1)

    def gather(g, b, sem):
        return pltpu.make_async_copy(table_hbm.at[idx_v.at[g]], bufs[b], sem)

    gather(0, 0, sem0).start()
    gather(1, 1, sem1).start()

    def reduce_chunk(g, buf):
        for grp in range(GRPS):
            s = grp * NF
            a0 = buf[s, 0:L]
            a1 = buf[s, L:D]
            for j in range(1, NF):
                a0 = a0 + buf[s + j, 0:L]
                a1 = a1 + buf[s + j, L:D]
            orow = g * GRPS + grp
            out_v[orow, 0:L] = a0 * INV
            out_v[orow, L:D] = a1 * INV

    def body(i, carry):
        for b in range(2):
            g = 2 * i + b
            gather(g, b, sems[b]).wait()
            reduce_chunk(g, bufs[b])

            @pl.when(g < NCH - 2)
            def _():
                gather(g + 2, b, sems[b]).start()

        return carry

    lax.fori_loop(0, NCH // 2, body, 0)

    pltpu.sync_copy(out_v, out_hbm.at[pl.ds(base * GRPS, BPW * 2)])


def kernel(x1, x2, table):
    idx = jnp.concatenate(
        [x1.astype(jnp.int32), x2.astype(jnp.int32)], axis=1)
    idx = idx.reshape(B // CHUNK, ROWS)
    tail = table[NBLK * BLK + TAIL:].reshape(NTT * D)
    dense = _detranspose(table.T, tail)
    out = _emb_pool(idx, dense.reshape(VOCAB, D))
    return out.reshape(B, 2, D)


# speedup vs baseline: 2.0703x; 1.0876x over previous
import functools

import jax
import jax.numpy as jnp
from jax import lax
from jax.experimental import pallas as pl
from jax.experimental.pallas import tpu as pltpu
from jax.experimental.pallas import tpu_sc as plsc

VOCAB = 1000000
D = 32
B = 16384
NF = 26
FT = 2 * NF
L = 16

NC = 2
NS = 16
NW = NC * NS
BPW = B // NW

BLK = 1024
NBLK = 976
NVB = BLK // L
TAIL = 512
NTT = VOCAB - NBLK * BLK - TAIL

CHUNK = 2
ROWS = CHUNK * FT
NCH = BPW // CHUNK
GRPS = CHUNK * 2
INV = 1.0 / NF

_mesh = plsc.VectorSubcoreMesh(core_axis_name="c", subcore_axis_name="s")


@functools.partial(
    pl.kernel,
    mesh=_mesh,
    compiler_params=pltpu.CompilerParams(
        use_tc_tiling_on_sc=True, needs_layout_passes=False),
    out_type=jax.ShapeDtypeStruct((VOCAB * D,), jnp.float32),
    scratch_types=[
        pltpu.VMEM((D, BLK), jnp.float32),
        pltpu.VMEM((D, BLK), jnp.float32),
        pltpu.VMEM((BLK * D,), jnp.float32),
        pltpu.SemaphoreType.DMA,
        pltpu.SemaphoreType.DMA,
    ],
)
def _detranspose(tt_hbm, tail_hbm, dense_hbm, stage0, stage1, outb,
                 sin0, sin1):
    wid = lax.axis_index("s") * NC + lax.axis_index("c")
    base = wid * 30 + jnp.minimum(wid, 16)
    nblk = jnp.where(wid < 16, 31, 30)

    iota32 = lax.iota(jnp.int32, L) * D
    stages = (stage0, stage1)
    sins = (sin0, sin1)

    def fetch(g, b):
        return pltpu.make_async_copy(
            tt_hbm.at[:, pl.ds((base + g) * BLK, BLK)], stages[b], sins[b])

    fetch(0, 0).start()
    fetch(1, 1).start()

    def transpose_block(stage, nv, i0):
        def vgroup(v, carry):
            av = iota32 + v * (L * D)
            for d in range(D):
                x = stage[d, pl.ds(v * L, L)]
                plsc.store_scatter(outb, [av + d], x)
            return carry

        lax.fori_loop(0, nv, vgroup, 0)
        pltpu.sync_copy(
            outb.at[pl.ds(0, nv * L * D)],
            dense_hbm.at[pl.ds(i0 * D, nv * L * D)])

    def body(i, carry):
        for b in range(2):
            g = 2 * i + b

            @pl.when(g < nblk)
            def _():
                fetch(g, b).wait()
                transpose_block(stages[b], NVB, (base + g) * BLK)

                @pl.when(g + 2 < nblk)
                def _():
                    fetch(g + 2, b).start()

        return carry

    lax.fori_loop(0, 16, body, 0)

    @pl.when(wid == 31)
    def _():
        pltpu.sync_copy(
            tt_hbm.at[:, pl.ds(NBLK * BLK, TAIL)],
            stage0.at[:, pl.ds(0, TAIL)])
        transpose_block(stage0, TAIL // L, NBLK * BLK)

    @pl.when(wid == 30)
    def _():
        pltpu.sync_copy(
            tail_hbm, dense_hbm.at[pl.ds((NBLK * BLK + TAIL) * D, NTT * D)])


@functools.partial(
    pl.kernel,
    mesh=_mesh,
    compiler_params=pltpu.CompilerParams(use_tc_tiling_on_sc=False),
    out_type=jax.ShapeDtypeStruct((B * 2, D), jnp.float32),
    scratch_types=[
        pltpu.VMEM((NCH, ROWS), jnp.int32),
        pltpu.VMEM((ROWS, D), jnp.float32),
        pltpu.VMEM((ROWS, D), jnp.float32),
        pltpu.VMEM((BPW * 2, D), jnp.float32),
        pltpu.SemaphoreType.DMA,
        pltpu.SemaphoreType.DMA,
    ],
)
def _emb_pool(idx_hbm, table_hbm, out_hbm, idx_v, rows0, rows1, out_v,
              sem0, sem1):
    wid = lax.axis_index("s") * NC + lax.axis_index("c")
    base = wid * NCH

    pltpu.sync_copy(idx_hbm.at[pl.ds(base, NCH)], idx_v)

    bufs = (rows0, rows1)
    sems = (sem0, sem1)

    def gather(g, b, sem):
        return pltpu.make_async_copy(table_hbm.at[idx_v.at[g]], bufs[b], sem)

    gather(0, 0, sem0).start()
    gather(1, 1, sem1).start()

    def reduce_chunk(g, buf):
        for grp in range(GRPS):
            s = grp * NF
            a0 = buf[s, 0:L]
            a1 = buf[s, L:D]
            for j in range(1, NF):
                a0 = a0 + buf[s + j, 0:L]
                a1 = a1 + buf[s + j, L:D]
            orow = g * GRPS + grp
            out_v[orow, 0:L] = a0 * INV
            out_v[orow, L:D] = a1 * INV

    def body(i, carry):
        for b in range(2):
            g = 2 * i + b
            gather(g, b, sems[b]).wait()
            reduce_chunk(g, bufs[b])

            @pl.when(g < NCH - 2)
            def _():
                gather(g + 2, b, sems[b]).start()

        return carry

    lax.fori_loop(0, NCH // 2, body, 0)

    pltpu.sync_copy(out_v, out_hbm.at[pl.ds(base * GRPS, BPW * 2)])


def kernel(x1, x2, table):
    idx = jnp.concatenate(
        [x1.astype(jnp.int32), x2.astype(jnp.int32)], axis=1)
    idx = idx.reshape(B // CHUNK, ROWS)
    tail = table[NBLK * BLK + TAIL:].reshape(NTT * D)
    dense = _detranspose(table.T, tail)
    out = _emb_pool(idx, dense.reshape(VOCAB, D))
    return out.reshape(B, 2, D)


# speedup vs baseline: 2.4703x; 1.1932x over previous
import functools

import jax
import jax.numpy as jnp
from jax import lax
from jax.experimental import pallas as pl
from jax.experimental.pallas import tpu as pltpu
from jax.experimental.pallas import tpu_sc as plsc

VOCAB = 1000000
D = 32
B = 16384
NF = 26
FT = 2 * NF
L = 16

NC = 2
NS = 16
NW = NC * NS
BPW = B // NW

BLK = 1024
NBLK = 976
NVB = BLK // L
TAIL = 512
NTT = VOCAB - NBLK * BLK - TAIL

CHUNK = 2
ROWS = CHUNK * FT
NCH = BPW // CHUNK
GRPS = CHUNK * 2
INV = 1.0 / NF

_mesh = plsc.VectorSubcoreMesh(core_axis_name="c", subcore_axis_name="s")


@functools.partial(
    pl.kernel,
    mesh=_mesh,
    compiler_params=pltpu.CompilerParams(
        use_tc_tiling_on_sc=True, needs_layout_passes=False),
    out_type=jax.ShapeDtypeStruct((VOCAB * D,), jnp.float32),
    scratch_types=[
        pltpu.VMEM((D, BLK), jnp.float32),
        pltpu.VMEM((D, BLK), jnp.float32),
        pltpu.VMEM((BLK * D,), jnp.float32),
        pltpu.SemaphoreType.DMA,
        pltpu.SemaphoreType.DMA,
    ],
)
def _detranspose(tt_hbm, tail_hbm, dense_hbm, stage0, stage1, outb,
                 sin0, sin1):
    wid = lax.axis_index("s") * NC + lax.axis_index("c")
    base = wid * 30 + jnp.minimum(wid, 16)
    nblk = jnp.where(wid < 16, 31, 30)

    iota32 = lax.iota(jnp.int32, L) * D
    stages = (stage0, stage1)
    sins = (sin0, sin1)

    def fetch(g, b):
        return pltpu.make_async_copy(
            tt_hbm.at[:, pl.ds((base + g) * BLK, BLK)], stages[b], sins[b])

    fetch(0, 0).start()
    fetch(1, 1).start()

    def transpose_block(stage, nv, i0):
        @plsc.parallel_loop(0, nv, unroll=2)
        def vgroup(v):
            av = iota32 + v * (L * D)
            for d in range(D):
                x = stage[d, pl.ds(v * L, L)]
                plsc.store_scatter(outb, [av + d], x)
        pltpu.sync_copy(
            outb.at[pl.ds(0, nv * L * D)],
            dense_hbm.at[pl.ds(i0 * D, nv * L * D)])

    def body(i, carry):
        for b in range(2):
            g = 2 * i + b

            @pl.when(g < nblk)
            def _():
                fetch(g, b).wait()
                transpose_block(stages[b], NVB, (base + g) * BLK)

                @pl.when(g + 2 < nblk)
                def _():
                    fetch(g + 2, b).start()

        return carry

    lax.fori_loop(0, 16, body, 0)

    @pl.when(wid == 31)
    def _():
        pltpu.sync_copy(
            tt_hbm.at[:, pl.ds(NBLK * BLK, TAIL)],
            stage0.at[:, pl.ds(0, TAIL)])
        transpose_block(stage0, TAIL // L, NBLK * BLK)

    @pl.when(wid == 30)
    def _():
        pltpu.sync_copy(
            tail_hbm, dense_hbm.at[pl.ds((NBLK * BLK + TAIL) * D, NTT * D)])


@functools.partial(
    pl.kernel,
    mesh=_mesh,
    compiler_params=pltpu.CompilerParams(use_tc_tiling_on_sc=False),
    out_type=jax.ShapeDtypeStruct((B * 2, D), jnp.float32),
    scratch_types=[
        pltpu.VMEM((NCH, ROWS), jnp.int32),
        pltpu.VMEM((ROWS, D), jnp.float32),
        pltpu.VMEM((ROWS, D), jnp.float32),
        pltpu.VMEM((BPW * 2, D), jnp.float32),
        pltpu.SemaphoreType.DMA,
        pltpu.SemaphoreType.DMA,
    ],
)
def _emb_pool(idx_hbm, table_hbm, out_hbm, idx_v, rows0, rows1, out_v,
              sem0, sem1):
    wid = lax.axis_index("s") * NC + lax.axis_index("c")
    base = wid * NCH

    pltpu.sync_copy(idx_hbm.at[pl.ds(base, NCH)], idx_v)

    bufs = (rows0, rows1)
    sems = (sem0, sem1)

    def gather(g, b, sem):
        return pltpu.make_async_copy(table_hbm.at[idx_v.at[g]], bufs[b], sem)

    gather(0, 0, sem0).start()
    gather(1, 1, sem1).start()

    def reduce_chunk(g, buf):
        for grp in range(GRPS):
            s = grp * NF
            a0 = buf[s, 0:L]
            a1 = buf[s, L:D]
            for j in range(1, NF):
                a0 = a0 + buf[s + j, 0:L]
                a1 = a1 + buf[s + j, L:D]
            orow = g * GRPS + grp
            out_v[orow, 0:L] = a0 * INV
            out_v[orow, L:D] = a1 * INV

    def body(i, carry):
        for b in range(2):
            g = 2 * i + b
            gather(g, b, sems[b]).wait()
            reduce_chunk(g, bufs[b])

            @pl.when(g < NCH - 2)
            def _():
                gather(g + 2, b, sems[b]).start()

        return carry

    lax.fori_loop(0, NCH // 2, body, 0)

    pltpu.sync_copy(out_v, out_hbm.at[pl.ds(base * GRPS, BPW * 2)])


def kernel(x1, x2, table):
    idx = jnp.concatenate(
        [x1.astype(jnp.int32), x2.astype(jnp.int32)], axis=1)
    idx = idx.reshape(B // CHUNK, ROWS)
    tail = table[NBLK * BLK + TAIL:].reshape(NTT * D)
    dense = _detranspose(table.T, tail)
    out = _emb_pool(idx, dense.reshape(VOCAB, D))
    return out.reshape(B, 2, D)
